# Initial kernel scaffold; baseline (speedup 1.0000x reference)
import jax
import jax.numpy as jnp
from jax.experimental import pallas as pl


def kernel(x, edge_index, W1, b1, W2, b2):
    raise NotImplementedError("write your pallas kernel here")


# speedup vs baseline: 32.6007x; 32.6007x over previous
import functools

import jax
import jax.numpy as jnp
from jax import lax
from jax.experimental import pallas as pl
from jax.experimental.pallas import tpu as pltpu
from jax.experimental.pallas import tpu_sc as plsc

NC = 2
NS = 16
NW = NC * NS
CHUNK = 128


def _sc_mesh():
    return plsc.VectorSubcoreMesh(core_axis_name="c", subcore_axis_name="s")


def _make_deg_kernel(NPAD, NCH):
    PER_W = NCH * CHUNK
    NRH = NPAD // CHUNK

    @functools.partial(
        pl.kernel,
        out_type=jax.ShapeDtypeStruct((NW, NRH, CHUNK), jnp.float32),
        mesh=_sc_mesh(),
        scratch_types=[
            pltpu.VMEM((PER_W,), jnp.int32),
            pltpu.VMEM((NRH, CHUNK), jnp.float32),
        ],
        compiler_params=pltpu.CompilerParams(needs_layout_passes=False),
    )
    def deg_kernel(dst_hbm, zeros_hbm, out_hbm, dst_v, hist_v):
        cid = lax.axis_index("c")
        sid = lax.axis_index("s")
        w = cid * NS + sid
        pltpu.sync_copy(dst_hbm.at[w], dst_v)
        pltpu.sync_copy(zeros_hbm, hist_v)
        ones = jnp.ones((16,), jnp.float32)

        def body(i, carry):
            idx = dst_v[pl.ds(i * 16, 16)]
            plsc.addupdate_scatter(
                hist_v, [lax.shift_right_logical(idx, 7), idx & 127], ones)
            return carry

        lax.fori_loop(0, PER_W // 16, body, 0)
        pltpu.sync_copy(hist_v, out_hbm.at[w])

    return deg_kernel


def _make_edge_kernel(N, NPAD, NCH, C):
    RPT = NPAD // NS
    NBUF = 2
    NHALF = 2
    HCH = NCH // NHALF
    assert NCH % (NHALF * NBUF) == 0

    @functools.partial(
        pl.kernel,
        out_type=jax.ShapeDtypeStruct((NC, NPAD, C), jnp.float32),
        mesh=_sc_mesh(),
        scratch_types=[
            pltpu.VMEM((HCH, CHUNK), jnp.int32),
            pltpu.VMEM((HCH, CHUNK), jnp.int32),
            pltpu.VMEM((NBUF, CHUNK, C), jnp.float32),
            pltpu.VMEM_SHARED((NPAD, C), jnp.float32),
            [pltpu.SemaphoreType.DMA] * NBUF,
            [pltpu.SemaphoreType.DMA] * NBUF,
        ],
    )
    def edge_kernel(xs_hbm, src_hbm, dst_hbm, zeros_hbm, out_hbm,
                    src_v, dst_v, rows_v, acc, gsem, ssem):
        cid = lax.axis_index("c")
        sid = lax.axis_index("s")
        w = cid * NS + sid
        base = sid * RPT
        pltpu.sync_copy(zeros_hbm.at[pl.ds(base, RPT)], acc.at[pl.ds(base, RPT)])
        plsc.subcore_barrier()

        for h in range(NHALF):
            pltpu.sync_copy(src_hbm.at[w, pl.ds(h * HCH, HCH)], src_v)
            pltpu.sync_copy(dst_hbm.at[w, pl.ds(h * HCH, HCH)], dst_v)
            for b in range(NBUF):
                pltpu.async_copy(
                    xs_hbm.at[src_v.at[b]], rows_v.at[b], gsem[b])

            def body(g, carry):
                for b in range(NBUF):
                    j = g * NBUF + b
                    pltpu.make_async_copy(
                        xs_hbm.at[src_v.at[j]], rows_v.at[b], gsem[b]).wait()
                    pltpu.sync_copy(rows_v.at[b], acc.at[dst_v.at[j]],
                                    add=True)
                    pltpu.async_copy(
                        xs_hbm.at[src_v.at[j + NBUF]], rows_v.at[b], gsem[b])
                return carry

            lax.fori_loop(0, HCH // NBUF - 1, body, 0)
            for b in range(NBUF):
                j = HCH - NBUF + b
                pltpu.make_async_copy(
                    xs_hbm.at[src_v.at[j]], rows_v.at[b], gsem[b]).wait()
                pltpu.sync_copy(rows_v.at[b], acc.at[dst_v.at[j]], add=True)

        plsc.subcore_barrier()
        pltpu.sync_copy(acc.at[pl.ds(base, RPT)], out_hbm.at[cid, pl.ds(base, RPT)])

    return edge_kernel


def _rows_to_col(rows, BR):
    eye = (lax.broadcasted_iota(jnp.int32, (CHUNK, CHUNK), 0) ==
           lax.broadcasted_iota(jnp.int32, (CHUNK, CHUNK), 1))
    cols = []
    for g in range(BR):
        bc = jnp.broadcast_to(rows[g:g + 1, :], (CHUNK, CHUNK))
        cols.append(jnp.sum(jnp.where(eye, bc, 0.0), axis=1, keepdims=True))
    return jnp.concatenate(cols, axis=0)


def _k1(x, W1, degp, N, BN, C):
    BR = BN // CHUNK

    def body(x_ref, w_ref, degp_ref, xs_ref, dis_ref):
        deg = _rows_to_col(jnp.sum(degp_ref[...], axis=0), BR) + 1.0
        dis = lax.rsqrt(deg)
        xw = jnp.dot(x_ref[...], w_ref[...], preferred_element_type=jnp.float32)
        xs_ref[...] = xw * dis
        dis_ref[...] = dis

    return pl.pallas_call(
        body,
        grid=(-(-N // BN),),
        in_specs=[
            pl.BlockSpec((BN, C), lambda i: (i, 0)),
            pl.BlockSpec((C, C), lambda i: (0, 0)),
            pl.BlockSpec((NW, BR, CHUNK), lambda i: (0, i, 0)),
        ],
        out_specs=[
            pl.BlockSpec((BN, C), lambda i: (i, 0)),
            pl.BlockSpec((BN, 1), lambda i: (i, 0)),
        ],
        out_shape=[
            jax.ShapeDtypeStruct((N, C), jnp.float32),
            jax.ShapeDtypeStruct((N, 1), jnp.float32),
        ],
    )(x, W1, degp)


def _k2(accp, xs1, dis, b1, W2, N, BN, C):
    def body(accp_ref, xs1_ref, dis_ref, b_ref, w_ref, xs2_ref):
        acc = accp_ref[0] + accp_ref[1] + xs1_ref[...]
        h = jnp.maximum(acc * dis_ref[...] + b_ref[...], 0.0)
        xs2_ref[...] = jnp.dot(
            h, w_ref[...], preferred_element_type=jnp.float32) * dis_ref[...]

    return pl.pallas_call(
        body,
        grid=(-(-N // BN),),
        in_specs=[
            pl.BlockSpec((NC, BN, C), lambda i: (0, i, 0)),
            pl.BlockSpec((BN, C), lambda i: (i, 0)),
            pl.BlockSpec((BN, 1), lambda i: (i, 0)),
            pl.BlockSpec((1, C), lambda i: (0, 0)),
            pl.BlockSpec((C, C), lambda i: (0, 0)),
        ],
        out_specs=pl.BlockSpec((BN, C), lambda i: (i, 0)),
        out_shape=jax.ShapeDtypeStruct((N, C), jnp.float32),
    )(accp, xs1, dis, b1, W2)


def _k3(accp, xs2, dis, b2, N, BN, C):
    inv_n = 1.0 / N

    def body(accp_ref, xs2_ref, dis_ref, b_ref, h_ref, gs_ref):
        i = pl.program_id(0)
        acc = accp_ref[0] + accp_ref[1] + xs2_ref[...]
        h = jnp.maximum(acc * dis_ref[...] + b_ref[...], 0.0)
        h_ref[...] = h

        @pl.when(i == 0)
        def _():
            gs_ref[...] = jnp.zeros_like(gs_ref)

        row = i * BN + lax.broadcasted_iota(jnp.int32, (BN, 1), 0)
        hm = jnp.where(row < N, h, 0.0)
        gs_ref[...] += jnp.sum(hm, axis=0, keepdims=True) * inv_n

    return pl.pallas_call(
        body,
        grid=(-(-N // BN),),
        in_specs=[
            pl.BlockSpec((NC, BN, C), lambda i: (0, i, 0)),
            pl.BlockSpec((BN, C), lambda i: (i, 0)),
            pl.BlockSpec((BN, 1), lambda i: (i, 0)),
            pl.BlockSpec((1, C), lambda i: (0, 0)),
        ],
        out_specs=[
            pl.BlockSpec((BN, C), lambda i: (i, 0)),
            pl.BlockSpec((1, C), lambda i: (0, 0)),
        ],
        out_shape=[
            jax.ShapeDtypeStruct((N, C), jnp.float32),
            jax.ShapeDtypeStruct((1, C), jnp.float32),
        ],
    )(accp, xs2, dis, b2)


def kernel(x, edge_index, W1, b1, W2, b2):
    N, C = x.shape
    E = edge_index.shape[1]
    NCH = -(-(-(-E // (NW * CHUNK))) // 4) * 4
    EP = NW * NCH * CHUNK
    RPT = -(-(-(-(N + 1) // NS)) // 8) * 8
    NPAD = RPT * NS
    BN = 1024

    src = edge_index[0]
    dst = edge_index[1]
    pad_i = jnp.arange(EP - E, dtype=jnp.int32)
    srcp = jnp.concatenate([src, pad_i % N]).reshape(NW, NCH, CHUNK)
    dstp = jnp.concatenate(
        [dst, N + pad_i % (NPAD - N)]).reshape(NW, NCH, CHUNK)

    zeros_deg = jnp.zeros((NPAD // CHUNK, CHUNK), jnp.float32)
    zeros_acc = jnp.zeros((NPAD, C), jnp.float32)

    deg_k = _make_deg_kernel(NPAD, NCH)
    edge_k = _make_edge_kernel(N, NPAD, NCH, C)

    degp = deg_k(dstp.reshape(NW, NCH * CHUNK), zeros_deg)
    xs1, dis = _k1(x, W1, degp, N, BN, C)
    accp1 = edge_k(xs1, srcp, dstp, zeros_acc)
    xs2 = _k2(accp1, xs1, dis, b1.reshape(1, C), W2, N, BN, C)
    accp2 = edge_k(xs2, srcp, dstp, zeros_acc)
    h, gmean = _k3(accp2, xs2, dis, b2.reshape(1, C), N, BN, C)
    return (h, gmean)


# speedup vs baseline: 34.0615x; 1.0448x over previous
import functools

import jax
import jax.numpy as jnp
from jax import lax
from jax.experimental import pallas as pl
from jax.experimental.pallas import tpu as pltpu
from jax.experimental.pallas import tpu_sc as plsc

NC = 2
NS = 16
NW = NC * NS
CHUNK = 128


def _sc_mesh():
    return plsc.VectorSubcoreMesh(core_axis_name="c", subcore_axis_name="s")


def _make_deg_kernel(NPAD, NCH, E):
    PER_W = NCH * CHUNK
    NRH = NPAD // CHUNK
    TAIL0 = (NW - 1) * PER_W
    NTW = (E - TAIL0) // CHUNK
    TW0 = 6

    @functools.partial(
        pl.kernel,
        out_type=jax.ShapeDtypeStruct((NW, NRH, CHUNK), jnp.float32),
        mesh=_sc_mesh(),
        scratch_types=[
            pltpu.VMEM((PER_W + CHUNK,), jnp.int32),
            pltpu.VMEM((NRH, CHUNK), jnp.float32),
        ],
        compiler_params=pltpu.CompilerParams(needs_layout_passes=False),
    )
    def deg_kernel(ei_hbm, zeros_hbm, out_hbm, dst_v, hist_v):
        cid = lax.axis_index("c")
        sid = lax.axis_index("s")
        w = cid * NS + sid
        pltpu.sync_copy(zeros_hbm, hist_v)
        ones = jnp.ones((16,), jnp.float32)

        def body(i, carry):
            idx = dst_v[pl.ds(i * 16, 16)]
            plsc.addupdate_scatter(
                hist_v, [lax.shift_right_logical(idx, 7), idx & 127], ones)
            return carry

        @pl.when(w < NW - 1)
        def _():
            pltpu.sync_copy(ei_hbm.at[1, pl.ds(w * PER_W, PER_W)],
                            dst_v.at[pl.ds(0, PER_W)])
            lax.fori_loop(0, PER_W // 16, body, 0)

        @pl.when((w >= TW0) & (w < TW0 + NTW))
        def _():
            pltpu.sync_copy(
                ei_hbm.at[1, pl.ds(TAIL0 + (w - TW0) * CHUNK, CHUNK)],
                dst_v.at[pl.ds(PER_W, CHUNK)])
            lax.fori_loop(PER_W // 16, (PER_W + CHUNK) // 16, body, 0)

        pltpu.sync_copy(hist_v, out_hbm.at[w])

    return deg_kernel


def _make_edge_kernel(N, NPAD, NCH, C, E):
    RPT = NPAD // NS
    NBUF = 2
    NHALF = 2
    HCH = NCH // NHALF
    HW = HCH * CHUNK
    PER_W = NCH * CHUNK
    TAIL0 = (NW - 1) * PER_W
    NTW = (E - TAIL0) // CHUNK
    TW0 = 6
    assert NCH % (NHALF * NBUF) == 0

    @functools.partial(
        pl.kernel,
        out_type=jax.ShapeDtypeStruct((NC, NPAD, C), jnp.float32),
        mesh=_sc_mesh(),
        scratch_types=[
            pltpu.VMEM((HW,), jnp.int32),
            pltpu.VMEM((HW,), jnp.int32),
            pltpu.VMEM((NBUF, CHUNK, C), jnp.float32),
            pltpu.VMEM_SHARED((NPAD, C), jnp.float32),
            [pltpu.SemaphoreType.DMA] * NBUF,
            [pltpu.SemaphoreType.DMA] * NBUF,
        ],
    )
    def edge_kernel(xs_hbm, ei_hbm, zeros_hbm, out_hbm,
                    src_v, dst_v, rows_v, acc, gsem, ssem):
        cid = lax.axis_index("c")
        sid = lax.axis_index("s")
        w = cid * NS + sid
        base = sid * RPT
        pltpu.sync_copy(zeros_hbm.at[pl.ds(base, RPT)], acc.at[pl.ds(base, RPT)])
        plsc.subcore_barrier()

        def gidx(j):
            return src_v.at[pl.ds(j * CHUNK, CHUNK)]

        def sidx(j):
            return dst_v.at[pl.ds(j * CHUNK, CHUNK)]

        @pl.when(w < NW - 1)
        def _():
            for h in range(NHALF):
                off = w * PER_W + h * HW
                pltpu.sync_copy(ei_hbm.at[0, pl.ds(off, HW)], src_v)
                pltpu.sync_copy(ei_hbm.at[1, pl.ds(off, HW)], dst_v)
                for b in range(NBUF):
                    pltpu.async_copy(xs_hbm.at[gidx(b)], rows_v.at[b],
                                     gsem[b])

                def body(g, carry):
                    for b in range(NBUF):
                        j = g * NBUF + b
                        pltpu.make_async_copy(
                            xs_hbm.at[gidx(j)], rows_v.at[b], gsem[b]).wait()
                        pltpu.sync_copy(rows_v.at[b], acc.at[sidx(j)],
                                        add=True)
                        pltpu.async_copy(
                            xs_hbm.at[gidx(j + NBUF)], rows_v.at[b], gsem[b])
                    return carry

                lax.fori_loop(0, HCH // NBUF - 1, body, 0)
                for b in range(NBUF):
                    j = HCH - NBUF + b
                    pltpu.make_async_copy(
                        xs_hbm.at[gidx(j)], rows_v.at[b], gsem[b]).wait()
                    pltpu.sync_copy(rows_v.at[b], acc.at[sidx(j)], add=True)

        @pl.when((w >= TW0) & (w < TW0 + NTW))
        def _():
            toff = TAIL0 + (w - TW0) * CHUNK
            pltpu.sync_copy(ei_hbm.at[0, pl.ds(toff, CHUNK)],
                            src_v.at[pl.ds(0, CHUNK)])
            pltpu.sync_copy(ei_hbm.at[1, pl.ds(toff, CHUNK)],
                            dst_v.at[pl.ds(0, CHUNK)])
            pltpu.async_copy(xs_hbm.at[gidx(0)], rows_v.at[0], gsem[0]).wait()
            pltpu.sync_copy(rows_v.at[0], acc.at[sidx(0)], add=True)

        plsc.subcore_barrier()
        pltpu.sync_copy(acc.at[pl.ds(base, RPT)], out_hbm.at[cid, pl.ds(base, RPT)])

    return edge_kernel


def _rows_to_col(rows, BR):
    eye = (lax.broadcasted_iota(jnp.int32, (CHUNK, CHUNK), 0) ==
           lax.broadcasted_iota(jnp.int32, (CHUNK, CHUNK), 1))
    cols = []
    for g in range(BR):
        bc = jnp.broadcast_to(rows[g:g + 1, :], (CHUNK, CHUNK))
        cols.append(jnp.sum(jnp.where(eye, bc, 0.0), axis=1, keepdims=True))
    return jnp.concatenate(cols, axis=0)


def _k1(x, W1, degp, N, BN, C):
    BR = BN // CHUNK

    def body(x_ref, w_ref, degp_ref, xs_ref, dis_ref):
        deg = _rows_to_col(jnp.sum(degp_ref[...], axis=0), BR) + 1.0
        dis = lax.rsqrt(deg)
        xw = jnp.dot(x_ref[...], w_ref[...], preferred_element_type=jnp.float32)
        xs_ref[...] = xw * dis
        dis_ref[...] = dis

    return pl.pallas_call(
        body,
        grid=(-(-N // BN),),
        in_specs=[
            pl.BlockSpec((BN, C), lambda i: (i, 0)),
            pl.BlockSpec((C, C), lambda i: (0, 0)),
            pl.BlockSpec((NW, BR, CHUNK), lambda i: (0, i, 0)),
        ],
        out_specs=[
            pl.BlockSpec((BN, C), lambda i: (i, 0)),
            pl.BlockSpec((BN, 1), lambda i: (i, 0)),
        ],
        out_shape=[
            jax.ShapeDtypeStruct((N, C), jnp.float32),
            jax.ShapeDtypeStruct((N, 1), jnp.float32),
        ],
    )(x, W1, degp)


def _k2(accp, xs1, dis, b1, W2, N, BN, C):
    def body(accp_ref, xs1_ref, dis_ref, b_ref, w_ref, xs2_ref):
        acc = accp_ref[0] + accp_ref[1] + xs1_ref[...]
        h = jnp.maximum(acc * dis_ref[...] + b_ref[...], 0.0)
        xs2_ref[...] = jnp.dot(
            h, w_ref[...], preferred_element_type=jnp.float32) * dis_ref[...]

    return pl.pallas_call(
        body,
        grid=(-(-N // BN),),
        in_specs=[
            pl.BlockSpec((NC, BN, C), lambda i: (0, i, 0)),
            pl.BlockSpec((BN, C), lambda i: (i, 0)),
            pl.BlockSpec((BN, 1), lambda i: (i, 0)),
            pl.BlockSpec((1, C), lambda i: (0, 0)),
            pl.BlockSpec((C, C), lambda i: (0, 0)),
        ],
        out_specs=pl.BlockSpec((BN, C), lambda i: (i, 0)),
        out_shape=jax.ShapeDtypeStruct((N, C), jnp.float32),
    )(accp, xs1, dis, b1, W2)


def _k3(accp, xs2, dis, b2, N, BN, C):
    inv_n = 1.0 / N

    def body(accp_ref, xs2_ref, dis_ref, b_ref, h_ref, gs_ref):
        i = pl.program_id(0)
        acc = accp_ref[0] + accp_ref[1] + xs2_ref[...]
        h = jnp.maximum(acc * dis_ref[...] + b_ref[...], 0.0)
        h_ref[...] = h

        @pl.when(i == 0)
        def _():
            gs_ref[...] = jnp.zeros_like(gs_ref)

        row = i * BN + lax.broadcasted_iota(jnp.int32, (BN, 1), 0)
        hm = jnp.where(row < N, h, 0.0)
        gs_ref[...] += jnp.sum(hm, axis=0, keepdims=True) * inv_n

    return pl.pallas_call(
        body,
        grid=(-(-N // BN),),
        in_specs=[
            pl.BlockSpec((NC, BN, C), lambda i: (0, i, 0)),
            pl.BlockSpec((BN, C), lambda i: (i, 0)),
            pl.BlockSpec((BN, 1), lambda i: (i, 0)),
            pl.BlockSpec((1, C), lambda i: (0, 0)),
        ],
        out_specs=[
            pl.BlockSpec((BN, C), lambda i: (i, 0)),
            pl.BlockSpec((1, C), lambda i: (0, 0)),
        ],
        out_shape=[
            jax.ShapeDtypeStruct((N, C), jnp.float32),
            jax.ShapeDtypeStruct((1, C), jnp.float32),
        ],
    )(accp, xs2, dis, b2)


def kernel(x, edge_index, W1, b1, W2, b2):
    N, C = x.shape
    E = edge_index.shape[1]
    NCH = -(-(-(-E // (NW * CHUNK))) // 4) * 4
    assert (NW - 1) * NCH * CHUNK <= E and E % CHUNK == 0
    RPT = -(-(-(-(N + 1) // NS)) // 8) * 8
    NPAD = RPT * NS
    BN = 1024

    zeros_deg = jnp.zeros((NPAD // CHUNK, CHUNK), jnp.float32)
    zeros_acc = jnp.zeros((NPAD, C), jnp.float32)

    deg_k = _make_deg_kernel(NPAD, NCH, E)
    edge_k = _make_edge_kernel(N, NPAD, NCH, C, E)

    degp = deg_k(edge_index, zeros_deg)
    xs1, dis = _k1(x, W1, degp, N, BN, C)
    accp1 = edge_k(xs1, edge_index, zeros_acc)
    xs2 = _k2(accp1, xs1, dis, b1.reshape(1, C), W2, N, BN, C)
    accp2 = edge_k(xs2, edge_index, zeros_acc)
    h, gmean = _k3(accp2, xs2, dis, b2.reshape(1, C), N, BN, C)
    return (h, gmean)
